# baseline (speedup 1.0000x reference)
import functools

import jax
import jax.numpy as jnp
from jax import lax
from jax.experimental import pallas as pl
from jax.experimental.pallas import tpu as pltpu
from jax.experimental.pallas import tpu_sc as plsc

NC = 2
NS = 16
LANE = 16
SUB = 128
RING = 8
LOOK = 4
FCH = 200


def _strided(k, kmax, nchunk, fn):
    pass


def _lightgcn_body(uE, iE, emask, col2, row2, val, outU, outI, curA, curB,
                   curC, acc_sh, cidx_v, ridx_v, val_v, rows_v, gsem, ssem,
                   isem, *, n, nu, e, ch, zch):
    c = lax.axis_index("c")
    s = lax.axis_index("s")
    nsub = ch // SUB
    nblk = e // ch
    kmax = (nblk + NS - 1) // NS
    nzch = n // zch
    kzmax = (nzch + NS - 1) // NS
    nfch = n // FCH
    kfmax = (nfch + NS - 1) // NS
    ni = n - nu
    cofs = c * jnp.int32(n)

    def fire_idx(bid, p):
        brow = bid * nsub
        base = bid * ch
        pltpu.async_copy(col2.at[pl.ds(brow, nsub)], cidx_v.at[p], isem.at[p])
        pltpu.async_copy(row2.at[pl.ds(brow, nsub)], ridx_v.at[p], isem.at[p])
        pltpu.async_copy(val.at[pl.ds(base, ch)], val_v.at[p], isem.at[p])

    fire_idx(s, 0)

    def prep_body(k, _):
        cid = s + k * NS

        @pl.when(cid < nfch)
        def _():
            r0 = cid * FCH
            a = rows_v.at[pl.ds(0, FCH)]
            m = rows_v.at[pl.ds(256, FCH)]

            @pl.when(r0 < nu)
            def _():
                pltpu.sync_copy(uE.at[pl.ds(r0, FCH), pl.ds(c * LANE, LANE)], a)

            @pl.when(r0 >= nu)
            def _():
                pltpu.sync_copy(
                    iE.at[pl.ds(r0 - nu, FCH), pl.ds(c * LANE, LANE)], a
                )

            pltpu.sync_copy(emask.at[pl.ds(r0, FCH), pl.ds(c * LANE, LANE)], m)

            def mbody(i, _):
                a[i, :] = a[i, :] * m[i, :]
                return 0

            lax.fori_loop(0, FCH, mbody, 0)
            pltpu.sync_copy(a, curA.at[pl.ds(cofs + r0, FCH)])

        return 0

    lax.fori_loop(0, kfmax, prep_body, 0)

    zero = jnp.zeros((LANE,), jnp.float32)

    def zfill(i, _):
        rows_v[i, :] = zero
        return 0

    lax.fori_loop(0, zch, zfill, 0)

    def zcopy_body(k, _):
        cid = s + k * NS

        @pl.when(cid < nzch)
        def _():
            pltpu.sync_copy(
                rows_v.at[pl.ds(0, zch)], acc_sh.at[pl.ds(cid * zch, zch)]
            )

        return 0

    lax.fori_loop(0, kzmax, zcopy_body, 0)
    plsc.subcore_barrier()

    def wait_idx(p):
        pltpu.make_async_copy(col2.at[pl.ds(0, nsub)], cidx_v.at[p],
                              isem.at[p]).wait()
        pltpu.make_async_copy(row2.at[pl.ds(0, nsub)], ridx_v.at[p],
                              isem.at[p]).wait()
        pltpu.make_async_copy(val.at[pl.ds(0, ch)], val_v.at[p],
                              isem.at[p]).wait()

    ntail = (e // SUB) % nsub
    tail_tile = NS - 1

    def run_layer(src, dst, last, prefired=False):
        def fire_gather(p, j):
            r = lax.rem(j, RING)
            pltpu.async_copy(src.at[cidx_v.at[p].at[j]],
                             rows_v.at[pl.ds(r * SUB, SUB)], gsem.at[r])

        def wait_gather(j):
            r = lax.rem(j, RING)
            pltpu.make_async_copy(src.at[cidx_v.at[0].at[0]],
                                  rows_v.at[pl.ds(r * SUB, SUB)],
                                  gsem.at[r]).wait()

        def fire_scatter(p, j):
            r = lax.rem(j, RING)
            pltpu.async_copy(rows_v.at[pl.ds(r * SUB, SUB)],
                             acc_sh.at[ridx_v.at[p].at[j]], ssem.at[r],
                             add=True)

        def wait_scatter(j):
            r = lax.rem(j, RING)
            pltpu.make_async_copy(rows_v.at[pl.ds(r * SUB, SUB)],
                                  acc_sh.at[ridx_v.at[0].at[0]],
                                  ssem.at[r]).wait()

        if not prefired:
            fire_idx(s, 0)

        def blk_body(k, _):
            bid = s + k * NS
            p = lax.rem(k, 2)

            @pl.when(bid < nblk)
            def _():
                wait_idx(p)

                @pl.when(c == 1)
                def _():
                    def abody(jj, _):
                        for l in range(SUB // LANE):
                            sl = pl.ds(l * LANE, LANE)
                            cidx_v[p, jj, sl] = cidx_v[p, jj, sl] + jnp.int32(n)
                        return 0

                    lax.fori_loop(0, nsub, abody, 0)

                bidn = bid + NS

                @pl.when(bidn < nblk)
                def _():
                    fire_idx(bidn, 1 - p)

                def prime(j, _):
                    fire_gather(p, j)
                    return 0

                lax.fori_loop(0, LOOK, prime, 0)

                def sub_body(j, _):
                    @pl.when(j >= LOOK)
                    def _():
                        wait_scatter(j - LOOK)

                    @pl.when(j + LOOK < nsub)
                    def _():
                        fire_gather(p, j + LOOK)

                    wait_gather(j)
                    r = lax.rem(j, RING)
                    for g in range(SUB // LANE):
                        vvec = val_v[p, pl.ds(j * SUB + g * LANE, LANE)]
                        for jj in range(LANE):
                            idx = r * SUB + g * LANE + jj
                            rows_v[idx, :] = rows_v[idx, :] * vvec[jj]
                    fire_scatter(p, j)
                    return 0

                lax.fori_loop(0, nsub, sub_body, 0)

                def drain(j, _):
                    wait_scatter(j)
                    return 0

                lax.fori_loop(nsub - LOOK, nsub, drain, 0)

            return 0

        lax.fori_loop(0, kmax, blk_body, 0)

        if ntail:
            @pl.when(s == tail_tile)
            def _():
                brow = nblk * nsub
                base = nblk * ch
                pltpu.sync_copy(col2.at[pl.ds(brow, ntail)],
                                cidx_v.at[0].at[pl.ds(0, ntail)])
                pltpu.sync_copy(row2.at[pl.ds(brow, ntail)],
                                ridx_v.at[0].at[pl.ds(0, ntail)])
                pltpu.sync_copy(val.at[pl.ds(base, ntail * SUB)],
                                val_v.at[0].at[pl.ds(0, ntail * SUB)])

                @pl.when(c == 1)
                def _():
                    def tbody(jj, _):
                        for l in range(SUB // LANE):
                            sl = pl.ds(l * LANE, LANE)
                            cidx_v[0, jj, sl] = cidx_v[0, jj, sl] + jnp.int32(n)
                        return 0

                    lax.fori_loop(0, ntail, tbody, 0)

                for j in range(ntail):
                    pltpu.async_copy(src.at[cidx_v.at[0].at[j]],
                                     rows_v.at[pl.ds(j * SUB, SUB)],
                                     gsem.at[j])
                for j in range(ntail):
                    pltpu.make_async_copy(src.at[cidx_v.at[0].at[0]],
                                          rows_v.at[pl.ds(j * SUB, SUB)],
                                          gsem.at[j]).wait()

                def tsbody(g, _):
                    vvec = val_v[0, pl.ds(g * LANE, LANE)]
                    for jj in range(LANE):
                        idx = g * LANE + jj
                        rows_v[idx, :] = rows_v[idx, :] * vvec[jj]
                    return 0

                lax.fori_loop(0, ntail * SUB // LANE, tsbody, 0)
                for j in range(ntail):
                    pltpu.sync_copy(rows_v.at[pl.ds(j * SUB, SUB)],
                                    acc_sh.at[ridx_v.at[0].at[j]], add=True)

        plsc.subcore_barrier()

        if not last:
            def zfill2(i, _):
                rows_v[i, :] = zero
                return 0

            lax.fori_loop(0, zch, zfill2, 0)

            def ecopy_body(k, _):
                cid = s + k * NS

                @pl.when(cid < nzch)
                def _():
                    r0 = cid * zch
                    pltpu.sync_copy(acc_sh.at[pl.ds(r0, zch)],
                                    dst.at[pl.ds(cofs + r0, zch)])
                    pltpu.sync_copy(rows_v.at[pl.ds(0, zch)],
                                    acc_sh.at[pl.ds(r0, zch)])

                return 0

            lax.fori_loop(0, kzmax, ecopy_body, 0)
            plsc.subcore_barrier()

    run_layer(curA, curB, last=False, prefired=True)
    run_layer(curB, curC, last=False)
    run_layer(curC, None, last=True)

    def fin_body(k, _):
        cid = s + k * NS

        @pl.when(cid < nfch)
        def _():
            r0 = cid * FCH
            a = rows_v.at[pl.ds(0, FCH)]
            b = rows_v.at[pl.ds(256, FCH)]
            d = rows_v.at[pl.ds(512, FCH)]
            t = rows_v.at[pl.ds(768, FCH)]
            pltpu.sync_copy(curA.at[pl.ds(cofs + r0, FCH)], a)
            pltpu.sync_copy(curB.at[pl.ds(cofs + r0, FCH)], b)
            pltpu.sync_copy(curC.at[pl.ds(cofs + r0, FCH)], d)
            pltpu.sync_copy(acc_sh.at[pl.ds(r0, FCH)], t)

            def abody(i, _):
                t[i, :] = (a[i, :] + b[i, :]) + (d[i, :] + t[i, :])
                return 0

            lax.fori_loop(0, FCH, abody, 0)

            @pl.when(r0 < nu)
            def _():
                pltpu.sync_copy(
                    t, outU.at[pl.ds(r0, FCH), pl.ds(c * LANE, LANE)]
                )

            @pl.when(r0 >= nu)
            def _():
                pltpu.sync_copy(
                    t, outI.at[pl.ds(r0 - nu, FCH), pl.ds(c * LANE, LANE)]
                )

        return 0

    lax.fori_loop(0, kfmax, fin_body, 0)


@functools.partial(jax.jit, static_argnames=("n", "nu", "e", "ch", "zch"))
def _lightgcn_call(uE, iE, emask, col2, row2, val, *, n, nu, e, ch, zch):
    mesh = plsc.VectorSubcoreMesh(
        core_axis_name="c", subcore_axis_name="s", num_cores=NC, num_subcores=NS
    )
    d = NC * LANE
    body = functools.partial(_lightgcn_body, n=n, nu=nu, e=e, ch=ch, zch=zch)
    return pl.kernel(
        body,
        out_type=(
            jax.ShapeDtypeStruct((nu, d), jnp.float32),
            jax.ShapeDtypeStruct((n - nu, d), jnp.float32),
            jax.ShapeDtypeStruct((NC * n, LANE), jnp.float32),
            jax.ShapeDtypeStruct((NC * n, LANE), jnp.float32),
            jax.ShapeDtypeStruct((NC * n, LANE), jnp.float32),
        ),
        mesh=mesh,
        scratch_types=[
            pltpu.VMEM_SHARED((n, LANE), jnp.float32),
            pltpu.VMEM((2, ch // SUB, SUB), jnp.int32),
            pltpu.VMEM((2, ch // SUB, SUB), jnp.int32),
            pltpu.VMEM((2, ch), jnp.float32),
            pltpu.VMEM((RING * SUB, LANE), jnp.float32),
            pltpu.SemaphoreType.DMA((RING,)),
            pltpu.SemaphoreType.DMA((RING,)),
            pltpu.SemaphoreType.DMA((2,)),
        ],
        compiler_params=pltpu.CompilerParams(use_tc_tiling_on_sc=False),
    )(uE, iE, emask, col2, row2, val)


def kernel(edge_index, edge_values, uEmbeds, iEmbeds, adj_mask1, adj_mask2,
           emb_mask2):
    ch, zch = 2048, 1000
    nu = uEmbeds.shape[0]
    n = nu + iEmbeds.shape[0]
    e = edge_values.shape[0]
    assert uEmbeds.shape[1] == NC * LANE and ch % SUB == 0
    assert n % zch == 0 and zch % 8 == 0 and zch <= RING * SUB
    assert n % FCH == 0 and nu % FCH == 0 and FCH % 8 == 0 and 4 * FCH <= RING * SUB

    assert e % SUB == 0
    val = edge_values * adj_mask1 * adj_mask2
    col2 = edge_index[1].reshape(e // SUB, SUB)
    row2 = edge_index[0].reshape(e // SUB, SUB)

    outU, outI, _, _, _ = _lightgcn_call(
        uEmbeds, iEmbeds, emb_mask2, col2, row2, val, n=n, nu=nu, e=e, ch=ch,
        zch=zch
    )
    return outU, outI


# speedup vs baseline: 1.1040x; 1.1040x over previous
import functools

import jax
import jax.numpy as jnp
from jax import lax
from jax.experimental import pallas as pl
from jax.experimental.pallas import tpu as pltpu
from jax.experimental.pallas import tpu_sc as plsc

NC = 2
NS = 16
LANE = 16
SUB = 128
RING = 8
LOOK = 4
FCH = 200


def _strided(k, kmax, nchunk, fn):
    pass


def _lightgcn_body(uE, iE, emask, col2, row2, val, outU, outI, curA, curB,
                   curC, acc_sh, cidx_v, ridx_v, val_v, rows_v, gsem, ssem,
                   isem, *, n, nu, e, ch, zch):
    c = lax.axis_index("c")
    s = lax.axis_index("s")
    nsub = ch // SUB
    nblk = e // ch
    kmax = (nblk + NS - 1) // NS
    nzch = n // zch
    kzmax = (nzch + NS - 1) // NS
    nfch = n // FCH
    kfmax = (nfch + NS - 1) // NS
    ni = n - nu
    cofs = c * jnp.int32(n)

    def fire_idx(bid, p):
        brow = bid * nsub
        base = bid * ch
        pltpu.async_copy(col2.at[pl.ds(brow, nsub)], cidx_v.at[p], isem.at[p])
        pltpu.async_copy(row2.at[pl.ds(brow, nsub)], ridx_v.at[p], isem.at[p])
        pltpu.async_copy(val.at[pl.ds(base, ch)], val_v.at[p], isem.at[p])

    fire_idx(s, 0)

    def prep_body(k, _):
        cid = s + k * NS

        @pl.when(cid < nfch)
        def _():
            r0 = cid * FCH
            a = rows_v.at[pl.ds(0, FCH)]
            m = rows_v.at[pl.ds(256, FCH)]

            @pl.when(r0 < nu)
            def _():
                pltpu.sync_copy(uE.at[pl.ds(r0, FCH), pl.ds(c * LANE, LANE)], a)

            @pl.when(r0 >= nu)
            def _():
                pltpu.sync_copy(
                    iE.at[pl.ds(r0 - nu, FCH), pl.ds(c * LANE, LANE)], a
                )

            pltpu.sync_copy(emask.at[pl.ds(r0, FCH), pl.ds(c * LANE, LANE)], m)

            def mbody(i, _):
                a[i, :] = a[i, :] * m[i, :]
                return 0

            lax.fori_loop(0, FCH, mbody, 0)
            pltpu.sync_copy(a, curA.at[pl.ds(cofs + r0, FCH)])

        return 0

    lax.fori_loop(0, kfmax, prep_body, 0)

    zero = jnp.zeros((LANE,), jnp.float32)

    def zfill(i, _):
        rows_v[i, :] = zero
        return 0

    lax.fori_loop(0, zch, zfill, 0)

    def zcopy_body(k, _):
        cid = s + k * NS

        @pl.when(cid < nzch)
        def _():
            pltpu.sync_copy(
                rows_v.at[pl.ds(0, zch)], acc_sh.at[pl.ds(cid * zch, zch)]
            )

        return 0

    lax.fori_loop(0, kzmax, zcopy_body, 0)
    plsc.subcore_barrier()

    def wait_idx(p):
        pltpu.make_async_copy(col2.at[pl.ds(0, nsub)], cidx_v.at[p],
                              isem.at[p]).wait()
        pltpu.make_async_copy(row2.at[pl.ds(0, nsub)], ridx_v.at[p],
                              isem.at[p]).wait()
        pltpu.make_async_copy(val.at[pl.ds(0, ch)], val_v.at[p],
                              isem.at[p]).wait()

    ntail = (e // SUB) % nsub
    tail_tile = NS - 1

    def run_layer(src, dst, last, prefired=False):
        def fire_gather(p, j):
            r = lax.rem(j, RING)
            pltpu.async_copy(src.at[cidx_v.at[p].at[j]],
                             rows_v.at[pl.ds(r * SUB, SUB)], gsem.at[r])

        def wait_gather(j):
            r = lax.rem(j, RING)
            pltpu.make_async_copy(src.at[cidx_v.at[0].at[0]],
                                  rows_v.at[pl.ds(r * SUB, SUB)],
                                  gsem.at[r]).wait()

        def fire_scatter(p, j):
            r = lax.rem(j, RING)
            pltpu.async_copy(rows_v.at[pl.ds(r * SUB, SUB)],
                             acc_sh.at[ridx_v.at[p].at[j]], ssem.at[r],
                             add=True)

        def wait_scatter(j):
            r = lax.rem(j, RING)
            pltpu.make_async_copy(rows_v.at[pl.ds(r * SUB, SUB)],
                                  acc_sh.at[ridx_v.at[0].at[0]],
                                  ssem.at[r]).wait()

        if not prefired:
            fire_idx(s, 0)

        def blk_body(k, _):
            bid = s + k * NS
            p = lax.rem(k, 2)

            @pl.when(bid < nblk)
            def _():
                bidn = bid + NS

                @pl.when(bidn < nblk)
                def _():
                    fire_idx(bidn, 1 - p)

                def sub_body(j, _):
                    @pl.when(j >= LOOK)
                    def _():
                        wait_scatter(j - LOOK)

                    @pl.when(j + LOOK < nsub)
                    def _():
                        fire_gather(p, j + LOOK)

                    wait_gather(j)
                    r = lax.rem(j, RING)

                    def sbody(g, _):
                        vvec = val_v[p, pl.ds(j * SUB + g * LANE, LANE)]
                        for jj in range(LANE):
                            idx = r * SUB + g * LANE + jj
                            rows_v[idx, :] = rows_v[idx, :] * vvec[jj]
                        return 0

                    lax.fori_loop(0, SUB // LANE, sbody, 0)
                    fire_scatter(p, j)
                    return 0

                lax.fori_loop(0, nsub, sub_body, 0)

                @pl.when(bidn < nblk)
                def _():
                    stage_block(1 - p)

                    def prime(j, _):
                        fire_gather(1 - p, j)
                        return 0

                    lax.fori_loop(0, LOOK, prime, 0)

                def drain(j, _):
                    wait_scatter(j)
                    return 0

                lax.fori_loop(nsub - LOOK, nsub, drain, 0)

            return 0

        def stage_block(p):
            wait_idx(p)

            @pl.when(c == 1)
            def _():
                def abody(jj, _):
                    for l in range(SUB // LANE):
                        sl = pl.ds(l * LANE, LANE)
                        cidx_v[p, jj, sl] = cidx_v[p, jj, sl] + jnp.int32(n)
                    return 0

                lax.fori_loop(0, nsub, abody, 0)

        stage_block(0)

        def prime0(j, _):
            fire_gather(0, j)
            return 0

        lax.fori_loop(0, LOOK, prime0, 0)
        lax.fori_loop(0, kmax, blk_body, 0)

        if ntail:
            @pl.when(s == tail_tile)
            def _():
                brow = nblk * nsub
                base = nblk * ch
                pltpu.sync_copy(col2.at[pl.ds(brow, ntail)],
                                cidx_v.at[0].at[pl.ds(0, ntail)])
                pltpu.sync_copy(row2.at[pl.ds(brow, ntail)],
                                ridx_v.at[0].at[pl.ds(0, ntail)])
                pltpu.sync_copy(val.at[pl.ds(base, ntail * SUB)],
                                val_v.at[0].at[pl.ds(0, ntail * SUB)])

                @pl.when(c == 1)
                def _():
                    def tbody(jj, _):
                        for l in range(SUB // LANE):
                            sl = pl.ds(l * LANE, LANE)
                            cidx_v[0, jj, sl] = cidx_v[0, jj, sl] + jnp.int32(n)
                        return 0

                    lax.fori_loop(0, ntail, tbody, 0)

                for j in range(ntail):
                    pltpu.async_copy(src.at[cidx_v.at[0].at[j]],
                                     rows_v.at[pl.ds(j * SUB, SUB)],
                                     gsem.at[j])
                for j in range(ntail):
                    pltpu.make_async_copy(src.at[cidx_v.at[0].at[0]],
                                          rows_v.at[pl.ds(j * SUB, SUB)],
                                          gsem.at[j]).wait()

                def tsbody(g, _):
                    vvec = val_v[0, pl.ds(g * LANE, LANE)]
                    for jj in range(LANE):
                        idx = g * LANE + jj
                        rows_v[idx, :] = rows_v[idx, :] * vvec[jj]
                    return 0

                lax.fori_loop(0, ntail * SUB // LANE, tsbody, 0)
                for j in range(ntail):
                    pltpu.sync_copy(rows_v.at[pl.ds(j * SUB, SUB)],
                                    acc_sh.at[ridx_v.at[0].at[j]], add=True)

        plsc.subcore_barrier()

        if not last:
            def zfill2(i, _):
                rows_v[i, :] = zero
                return 0

            lax.fori_loop(0, zch, zfill2, 0)

            def ecopy_body(k, _):
                cid = s + k * NS

                @pl.when(cid < nzch)
                def _():
                    r0 = cid * zch
                    pltpu.sync_copy(acc_sh.at[pl.ds(r0, zch)],
                                    dst.at[pl.ds(cofs + r0, zch)])
                    pltpu.sync_copy(rows_v.at[pl.ds(0, zch)],
                                    acc_sh.at[pl.ds(r0, zch)])

                return 0

            lax.fori_loop(0, kzmax, ecopy_body, 0)
            plsc.subcore_barrier()

    run_layer(curA, curB, last=False, prefired=True)
    run_layer(curB, curC, last=False)
    run_layer(curC, None, last=True)

    def fin_body(k, _):
        cid = s + k * NS

        @pl.when(cid < nfch)
        def _():
            r0 = cid * FCH
            a = rows_v.at[pl.ds(0, FCH)]
            b = rows_v.at[pl.ds(256, FCH)]
            d = rows_v.at[pl.ds(512, FCH)]
            t = rows_v.at[pl.ds(768, FCH)]
            pltpu.sync_copy(curA.at[pl.ds(cofs + r0, FCH)], a)
            pltpu.sync_copy(curB.at[pl.ds(cofs + r0, FCH)], b)
            pltpu.sync_copy(curC.at[pl.ds(cofs + r0, FCH)], d)
            pltpu.sync_copy(acc_sh.at[pl.ds(r0, FCH)], t)

            def abody(i, _):
                t[i, :] = (a[i, :] + b[i, :]) + (d[i, :] + t[i, :])
                return 0

            lax.fori_loop(0, FCH, abody, 0)

            @pl.when(r0 < nu)
            def _():
                pltpu.sync_copy(
                    t, outU.at[pl.ds(r0, FCH), pl.ds(c * LANE, LANE)]
                )

            @pl.when(r0 >= nu)
            def _():
                pltpu.sync_copy(
                    t, outI.at[pl.ds(r0 - nu, FCH), pl.ds(c * LANE, LANE)]
                )

        return 0

    lax.fori_loop(0, kfmax, fin_body, 0)


@functools.partial(jax.jit, static_argnames=("n", "nu", "e", "ch", "zch"))
def _lightgcn_call(uE, iE, emask, col2, row2, val, *, n, nu, e, ch, zch):
    mesh = plsc.VectorSubcoreMesh(
        core_axis_name="c", subcore_axis_name="s", num_cores=NC, num_subcores=NS
    )
    d = NC * LANE
    body = functools.partial(_lightgcn_body, n=n, nu=nu, e=e, ch=ch, zch=zch)
    return pl.kernel(
        body,
        out_type=(
            jax.ShapeDtypeStruct((nu, d), jnp.float32),
            jax.ShapeDtypeStruct((n - nu, d), jnp.float32),
            jax.ShapeDtypeStruct((NC * n, LANE), jnp.float32),
            jax.ShapeDtypeStruct((NC * n, LANE), jnp.float32),
            jax.ShapeDtypeStruct((NC * n, LANE), jnp.float32),
        ),
        mesh=mesh,
        scratch_types=[
            pltpu.VMEM_SHARED((n, LANE), jnp.float32),
            pltpu.VMEM((2, ch // SUB, SUB), jnp.int32),
            pltpu.VMEM((2, ch // SUB, SUB), jnp.int32),
            pltpu.VMEM((2, ch), jnp.float32),
            pltpu.VMEM((RING * SUB, LANE), jnp.float32),
            pltpu.SemaphoreType.DMA((RING,)),
            pltpu.SemaphoreType.DMA((RING,)),
            pltpu.SemaphoreType.DMA((2,)),
        ],
        compiler_params=pltpu.CompilerParams(use_tc_tiling_on_sc=False),
    )(uE, iE, emask, col2, row2, val)


def kernel(edge_index, edge_values, uEmbeds, iEmbeds, adj_mask1, adj_mask2,
           emb_mask2):
    ch, zch = 2048, 1000
    nu = uEmbeds.shape[0]
    n = nu + iEmbeds.shape[0]
    e = edge_values.shape[0]
    assert uEmbeds.shape[1] == NC * LANE and ch % SUB == 0
    assert n % zch == 0 and zch % 8 == 0 and zch <= RING * SUB
    assert n % FCH == 0 and nu % FCH == 0 and FCH % 8 == 0 and 4 * FCH <= RING * SUB

    assert e % SUB == 0
    val = edge_values * adj_mask1 * adj_mask2
    col2 = edge_index[1].reshape(e // SUB, SUB)
    row2 = edge_index[0].reshape(e // SUB, SUB)

    outU, outI, _, _, _ = _lightgcn_call(
        uEmbeds, iEmbeds, emb_mask2, col2, row2, val, n=n, nu=nu, e=e, ch=ch,
        zch=zch
    )
    return outU, outI


# speedup vs baseline: 1.1041x; 1.0000x over previous
import functools

import jax
import jax.numpy as jnp
from jax import lax
from jax.experimental import pallas as pl
from jax.experimental.pallas import tpu as pltpu
from jax.experimental.pallas import tpu_sc as plsc

NC = 2
NS = 16
LANE = 16
SUB = 128
RING = 8
LOOK = 4
FCH = 200


def _strided(k, kmax, nchunk, fn):
    pass


def _lightgcn_body(uE, iE, emask, col2, row2, val, outU, outI, curA, curB,
                   curC, acc_sh, cidx_v, ridx_v, val_v, rows_v, gsem, ssem,
                   isem, *, n, nu, e, ch, zch):
    c = lax.axis_index("c")
    s = lax.axis_index("s")
    nsub = ch // SUB
    nblk = e // ch
    kmax = (nblk + NS - 1) // NS
    nzch = n // zch
    kzmax = (nzch + NS - 1) // NS
    nfch = n // FCH
    kfmax = (nfch + NS - 1) // NS
    ni = n - nu
    cofs = c * jnp.int32(n)

    def fire_idx(bid, p):
        brow = bid * nsub
        base = bid * ch
        pltpu.async_copy(col2.at[pl.ds(brow, nsub)], cidx_v.at[p], isem.at[p])
        pltpu.async_copy(row2.at[pl.ds(brow, nsub)], ridx_v.at[p], isem.at[p])
        pltpu.async_copy(val.at[pl.ds(base, ch)], val_v.at[p], isem.at[p])

    fire_idx(s, 0)

    def prep_body(k, _):
        cid = s + k * NS

        @pl.when(cid < nfch)
        def _():
            r0 = cid * FCH
            a = rows_v.at[pl.ds(0, FCH)]
            m = rows_v.at[pl.ds(256, FCH)]

            @pl.when(r0 < nu)
            def _():
                pltpu.sync_copy(uE.at[pl.ds(r0, FCH), pl.ds(c * LANE, LANE)], a)

            @pl.when(r0 >= nu)
            def _():
                pltpu.sync_copy(
                    iE.at[pl.ds(r0 - nu, FCH), pl.ds(c * LANE, LANE)], a
                )

            pltpu.sync_copy(emask.at[pl.ds(r0, FCH), pl.ds(c * LANE, LANE)], m)

            def mbody(i, _):
                a[i, :] = a[i, :] * m[i, :]
                return 0

            lax.fori_loop(0, FCH, mbody, 0)
            pltpu.sync_copy(a, curA.at[pl.ds(cofs + r0, FCH)])

        return 0

    lax.fori_loop(0, kfmax, prep_body, 0)

    zero = jnp.zeros((LANE,), jnp.float32)

    def zfill(i, _):
        rows_v[i, :] = zero
        return 0

    lax.fori_loop(0, zch, zfill, 0)

    def zcopy_body(k, _):
        cid = s + k * NS

        @pl.when(cid < nzch)
        def _():
            pltpu.sync_copy(
                rows_v.at[pl.ds(0, zch)], acc_sh.at[pl.ds(cid * zch, zch)]
            )

        return 0

    lax.fori_loop(0, kzmax, zcopy_body, 0)
    plsc.subcore_barrier()

    def wait_idx(p):
        pltpu.make_async_copy(col2.at[pl.ds(0, nsub)], cidx_v.at[p],
                              isem.at[p]).wait()
        pltpu.make_async_copy(row2.at[pl.ds(0, nsub)], ridx_v.at[p],
                              isem.at[p]).wait()
        pltpu.make_async_copy(val.at[pl.ds(0, ch)], val_v.at[p],
                              isem.at[p]).wait()

    ntail = (e // SUB) % nsub
    tail_tile = NS - 1

    def run_layer(src, dst, last, prefired=False):
        def fire_gather(p, j):
            r = lax.rem(j, RING)
            pltpu.async_copy(src.at[cidx_v.at[p].at[j]],
                             rows_v.at[pl.ds(r * SUB, SUB)], gsem.at[r])

        def wait_gather(j):
            r = lax.rem(j, RING)
            pltpu.make_async_copy(src.at[cidx_v.at[0].at[0]],
                                  rows_v.at[pl.ds(r * SUB, SUB)],
                                  gsem.at[r]).wait()

        def fire_scatter(p, j):
            r = lax.rem(j, RING)
            pltpu.async_copy(rows_v.at[pl.ds(r * SUB, SUB)],
                             acc_sh.at[ridx_v.at[p].at[j]], ssem.at[r],
                             add=True)

        def wait_scatter(j):
            r = lax.rem(j, RING)
            pltpu.make_async_copy(rows_v.at[pl.ds(r * SUB, SUB)],
                                  acc_sh.at[ridx_v.at[0].at[0]],
                                  ssem.at[r]).wait()

        if not prefired:
            fire_idx(s, 0)

        def blk_body(k, _):
            bid = s + k * NS
            p = lax.rem(k, 2)

            @pl.when(bid < nblk)
            def _():
                bidn = bid + NS

                @pl.when(bidn < nblk)
                def _():
                    fire_idx(bidn, 1 - p)

                def sub_body(j, _):
                    @pl.when(j >= LOOK)
                    def _():
                        wait_scatter(j - LOOK)

                    @pl.when(j + LOOK < nsub)
                    def _():
                        fire_gather(p, j + LOOK)

                    wait_gather(j)
                    r = lax.rem(j, RING)

                    def sbody(g, _):
                        vvec = val_v[p, pl.ds(j * SUB + g * LANE, LANE)]
                        for jj in range(LANE):
                            idx = r * SUB + g * LANE + jj
                            b = vvec[jax.lax.full((LANE,), jj, jnp.int32)]
                            rows_v[idx, :] = rows_v[idx, :] * b
                        return 0

                    lax.fori_loop(0, SUB // LANE, sbody, 0)
                    fire_scatter(p, j)
                    return 0

                lax.fori_loop(0, nsub, sub_body, 0)

                @pl.when(bidn < nblk)
                def _():
                    stage_block(1 - p)

                    def prime(j, _):
                        fire_gather(1 - p, j)
                        return 0

                    lax.fori_loop(0, LOOK, prime, 0)

                def drain(j, _):
                    wait_scatter(j)
                    return 0

                lax.fori_loop(nsub - LOOK, nsub, drain, 0)

            return 0

        def stage_block(p):
            wait_idx(p)

            @pl.when(c == 1)
            def _():
                def abody(jj, _):
                    for l in range(SUB // LANE):
                        sl = pl.ds(l * LANE, LANE)
                        cidx_v[p, jj, sl] = cidx_v[p, jj, sl] + jnp.int32(n)
                    return 0

                lax.fori_loop(0, nsub, abody, 0)

        stage_block(0)

        def prime0(j, _):
            fire_gather(0, j)
            return 0

        lax.fori_loop(0, LOOK, prime0, 0)
        lax.fori_loop(0, kmax, blk_body, 0)

        if ntail:
            @pl.when(s == tail_tile)
            def _():
                brow = nblk * nsub
                base = nblk * ch
                pltpu.sync_copy(col2.at[pl.ds(brow, ntail)],
                                cidx_v.at[0].at[pl.ds(0, ntail)])
                pltpu.sync_copy(row2.at[pl.ds(brow, ntail)],
                                ridx_v.at[0].at[pl.ds(0, ntail)])
                pltpu.sync_copy(val.at[pl.ds(base, ntail * SUB)],
                                val_v.at[0].at[pl.ds(0, ntail * SUB)])

                @pl.when(c == 1)
                def _():
                    def tbody(jj, _):
                        for l in range(SUB // LANE):
                            sl = pl.ds(l * LANE, LANE)
                            cidx_v[0, jj, sl] = cidx_v[0, jj, sl] + jnp.int32(n)
                        return 0

                    lax.fori_loop(0, ntail, tbody, 0)

                for j in range(ntail):
                    pltpu.async_copy(src.at[cidx_v.at[0].at[j]],
                                     rows_v.at[pl.ds(j * SUB, SUB)],
                                     gsem.at[j])
                for j in range(ntail):
                    pltpu.make_async_copy(src.at[cidx_v.at[0].at[0]],
                                          rows_v.at[pl.ds(j * SUB, SUB)],
                                          gsem.at[j]).wait()

                def tsbody(g, _):
                    vvec = val_v[0, pl.ds(g * LANE, LANE)]
                    for jj in range(LANE):
                        idx = g * LANE + jj
                        rows_v[idx, :] = rows_v[idx, :] * vvec[jj]
                    return 0

                lax.fori_loop(0, ntail * SUB // LANE, tsbody, 0)
                for j in range(ntail):
                    pltpu.sync_copy(rows_v.at[pl.ds(j * SUB, SUB)],
                                    acc_sh.at[ridx_v.at[0].at[j]], add=True)

        plsc.subcore_barrier()

        if not last:
            def zfill2(i, _):
                rows_v[i, :] = zero
                return 0

            lax.fori_loop(0, zch, zfill2, 0)

            def ecopy_body(k, _):
                cid = s + k * NS

                @pl.when(cid < nzch)
                def _():
                    r0 = cid * zch
                    pltpu.sync_copy(acc_sh.at[pl.ds(r0, zch)],
                                    dst.at[pl.ds(cofs + r0, zch)])
                    pltpu.sync_copy(rows_v.at[pl.ds(0, zch)],
                                    acc_sh.at[pl.ds(r0, zch)])

                return 0

            lax.fori_loop(0, kzmax, ecopy_body, 0)
            plsc.subcore_barrier()

    run_layer(curA, curB, last=False, prefired=True)
    run_layer(curB, curC, last=False)
    run_layer(curC, None, last=True)

    def fin_body(k, _):
        cid = s + k * NS

        @pl.when(cid < nfch)
        def _():
            r0 = cid * FCH
            a = rows_v.at[pl.ds(0, FCH)]
            b = rows_v.at[pl.ds(256, FCH)]
            d = rows_v.at[pl.ds(512, FCH)]
            t = rows_v.at[pl.ds(768, FCH)]
            pltpu.sync_copy(curA.at[pl.ds(cofs + r0, FCH)], a)
            pltpu.sync_copy(curB.at[pl.ds(cofs + r0, FCH)], b)
            pltpu.sync_copy(curC.at[pl.ds(cofs + r0, FCH)], d)
            pltpu.sync_copy(acc_sh.at[pl.ds(r0, FCH)], t)

            def abody(i, _):
                t[i, :] = (a[i, :] + b[i, :]) + (d[i, :] + t[i, :])
                return 0

            lax.fori_loop(0, FCH, abody, 0)

            @pl.when(r0 < nu)
            def _():
                pltpu.sync_copy(
                    t, outU.at[pl.ds(r0, FCH), pl.ds(c * LANE, LANE)]
                )

            @pl.when(r0 >= nu)
            def _():
                pltpu.sync_copy(
                    t, outI.at[pl.ds(r0 - nu, FCH), pl.ds(c * LANE, LANE)]
                )

        return 0

    lax.fori_loop(0, kfmax, fin_body, 0)


@functools.partial(jax.jit, static_argnames=("n", "nu", "e", "ch", "zch"))
def _lightgcn_call(uE, iE, emask, col2, row2, val, *, n, nu, e, ch, zch):
    mesh = plsc.VectorSubcoreMesh(
        core_axis_name="c", subcore_axis_name="s", num_cores=NC, num_subcores=NS
    )
    d = NC * LANE
    body = functools.partial(_lightgcn_body, n=n, nu=nu, e=e, ch=ch, zch=zch)
    return pl.kernel(
        body,
        out_type=(
            jax.ShapeDtypeStruct((nu, d), jnp.float32),
            jax.ShapeDtypeStruct((n - nu, d), jnp.float32),
            jax.ShapeDtypeStruct((NC * n, LANE), jnp.float32),
            jax.ShapeDtypeStruct((NC * n, LANE), jnp.float32),
            jax.ShapeDtypeStruct((NC * n, LANE), jnp.float32),
        ),
        mesh=mesh,
        scratch_types=[
            pltpu.VMEM_SHARED((n, LANE), jnp.float32),
            pltpu.VMEM((2, ch // SUB, SUB), jnp.int32),
            pltpu.VMEM((2, ch // SUB, SUB), jnp.int32),
            pltpu.VMEM((2, ch), jnp.float32),
            pltpu.VMEM((RING * SUB, LANE), jnp.float32),
            pltpu.SemaphoreType.DMA((RING,)),
            pltpu.SemaphoreType.DMA((RING,)),
            pltpu.SemaphoreType.DMA((2,)),
        ],
        compiler_params=pltpu.CompilerParams(use_tc_tiling_on_sc=False),
    )(uE, iE, emask, col2, row2, val)


def kernel(edge_index, edge_values, uEmbeds, iEmbeds, adj_mask1, adj_mask2,
           emb_mask2):
    ch, zch = 2048, 1000
    nu = uEmbeds.shape[0]
    n = nu + iEmbeds.shape[0]
    e = edge_values.shape[0]
    assert uEmbeds.shape[1] == NC * LANE and ch % SUB == 0
    assert n % zch == 0 and zch % 8 == 0 and zch <= RING * SUB
    assert n % FCH == 0 and nu % FCH == 0 and FCH % 8 == 0 and 4 * FCH <= RING * SUB

    assert e % SUB == 0
    val = edge_values * adj_mask1 * adj_mask2
    col2 = edge_index[1].reshape(e // SUB, SUB)
    row2 = edge_index[0].reshape(e // SUB, SUB)

    outU, outI, _, _, _ = _lightgcn_call(
        uEmbeds, iEmbeds, emb_mask2, col2, row2, val, n=n, nu=nu, e=e, ch=ch,
        zch=zch
    )
    return outU, outI


# speedup vs baseline: 1.1073x; 1.0029x over previous
import functools

import jax
import jax.numpy as jnp
from jax import lax
from jax.experimental import pallas as pl
from jax.experimental.pallas import tpu as pltpu
from jax.experimental.pallas import tpu_sc as plsc

NC = 2
NS = 16
LANE = 16
SUB = 128
RING = 8
LOOK = 4
FCH = 200


def _lightgcn_body(uE, iE, emask, col2, row2, val, outU, outI, curA, curB,
                   curC, acc_sh, cidx_v, ridx_v, val_v, rows_v, gsem, ssem,
                   isem, *, n, nu, e, ch, zch):
    c = lax.axis_index("c")
    s = lax.axis_index("s")
    nsub = ch // SUB
    nblk = e // ch
    kmax = (nblk + NS - 1) // NS
    nzch = n // zch
    kzmax = (nzch + NS - 1) // NS
    nfch = n // FCH
    kfmax = (nfch + NS - 1) // NS
    cofs = c * jnp.int32(n)

    def fire_idx(bid, p):
        brow = bid * nsub
        base = bid * ch
        pltpu.async_copy(col2.at[pl.ds(brow, nsub)], cidx_v.at[p], isem.at[p])
        pltpu.async_copy(row2.at[pl.ds(brow, nsub)], ridx_v.at[p], isem.at[p])
        pltpu.async_copy(val.at[pl.ds(base, ch)], val_v.at[p], isem.at[p])

    fire_idx(s, 0)

    def prep_body(k, _):
        cid = s + k * NS

        @pl.when(cid < nfch)
        def _():
            r0 = cid * FCH
            a = rows_v.at[pl.ds(0, FCH)]
            m = rows_v.at[pl.ds(256, FCH)]

            @pl.when(r0 < nu)
            def _():
                pltpu.sync_copy(uE.at[pl.ds(r0, FCH), pl.ds(c * LANE, LANE)], a)

            @pl.when(r0 >= nu)
            def _():
                pltpu.sync_copy(
                    iE.at[pl.ds(r0 - nu, FCH), pl.ds(c * LANE, LANE)], a
                )

            pltpu.sync_copy(emask.at[pl.ds(r0, FCH), pl.ds(c * LANE, LANE)], m)

            def mbody(i, _):
                a[i, :] = a[i, :] * m[i, :]
                return 0

            lax.fori_loop(0, FCH, mbody, 0)
            pltpu.sync_copy(a, curA.at[pl.ds(cofs + r0, FCH)])

        return 0

    lax.fori_loop(0, kfmax, prep_body, 0)

    zero = jnp.zeros((LANE,), jnp.float32)

    def zfill(i, _):
        rows_v[i, :] = zero
        return 0

    lax.fori_loop(0, zch, zfill, 0)

    def zcopy_body(k, _):
        cid = s + k * NS

        @pl.when(cid < nzch)
        def _():
            pltpu.sync_copy(
                rows_v.at[pl.ds(0, zch)], acc_sh.at[pl.ds(cid * zch, zch)]
            )

        return 0

    lax.fori_loop(0, kzmax, zcopy_body, 0)
    plsc.subcore_barrier()

    def wait_idx(p):
        pltpu.make_async_copy(col2.at[pl.ds(0, nsub)], cidx_v.at[p],
                              isem.at[p]).wait()
        pltpu.make_async_copy(row2.at[pl.ds(0, nsub)], ridx_v.at[p],
                              isem.at[p]).wait()
        pltpu.make_async_copy(val.at[pl.ds(0, ch)], val_v.at[p],
                              isem.at[p]).wait()

    ntail = (e // SUB) % nsub
    tail_tile = NS - 1

    def run_layer(src, dst, last, prefired=False):
        def fire_gather(p, j):
            r = lax.rem(j, RING)
            pltpu.async_copy(src.at[cidx_v.at[p].at[j]],
                             rows_v.at[pl.ds(r * SUB, SUB)], gsem.at[r])

        def wait_gather(j):
            r = lax.rem(j, RING)
            pltpu.make_async_copy(src.at[cidx_v.at[0].at[0]],
                                  rows_v.at[pl.ds(r * SUB, SUB)],
                                  gsem.at[r]).wait()

        def fire_scatter(p, j):
            r = lax.rem(j, RING)
            pltpu.async_copy(rows_v.at[pl.ds(r * SUB, SUB)],
                             acc_sh.at[ridx_v.at[p].at[j]], ssem.at[r],
                             add=True)

        def wait_scatter(j):
            r = lax.rem(j, RING)
            pltpu.make_async_copy(rows_v.at[pl.ds(r * SUB, SUB)],
                                  acc_sh.at[ridx_v.at[0].at[0]],
                                  ssem.at[r]).wait()

        if not prefired:
            fire_idx(s, 0)

        def blk_body(k, _):
            bid = s + k * NS
            p = lax.rem(k, 2)

            @pl.when(bid < nblk)
            def _():
                bidn = bid + NS

                @pl.when(bidn < nblk)
                def _():
                    fire_idx(bidn, 1 - p)

                def sub_body(j, _):
                    @pl.when(j >= LOOK)
                    def _():
                        wait_scatter(j - LOOK)

                    @pl.when(j + LOOK < nsub)
                    def _():
                        fire_gather(p, j + LOOK)

                    wait_gather(j)
                    r = lax.rem(j, RING)

                    def sbody(g, _):
                        vvec = val_v[p, pl.ds(j * SUB + g * LANE, LANE)]
                        for jj in range(LANE):
                            idx = r * SUB + g * LANE + jj
                            b = vvec[jax.lax.full((LANE,), jj, jnp.int32)]
                            rows_v[idx, :] = rows_v[idx, :] * b
                        return 0

                    lax.fori_loop(0, SUB // LANE, sbody, 0)
                    fire_scatter(p, j)
                    return 0

                lax.fori_loop(0, nsub, sub_body, 0)

                @pl.when(bidn < nblk)
                def _():
                    stage_block(1 - p)

                    def prime(j, _):
                        fire_gather(1 - p, j)
                        return 0

                    lax.fori_loop(0, LOOK, prime, 0)

                def drain(j, _):
                    wait_scatter(j)
                    return 0

                lax.fori_loop(nsub - LOOK, nsub, drain, 0)

            return 0

        def stage_block(p):
            wait_idx(p)

            @pl.when(c == 1)
            def _():
                def abody(jj, _):
                    for l in range(SUB // LANE):
                        sl = pl.ds(l * LANE, LANE)
                        cidx_v[p, jj, sl] = cidx_v[p, jj, sl] + jnp.int32(n)
                    return 0

                lax.fori_loop(0, nsub, abody, 0)

        stage_block(0)

        def prime0(j, _):
            fire_gather(0, j)
            return 0

        lax.fori_loop(0, LOOK, prime0, 0)
        lax.fori_loop(0, kmax, blk_body, 0)

        if ntail:
            @pl.when(s == tail_tile)
            def _():
                brow = nblk * nsub
                base = nblk * ch
                pltpu.sync_copy(col2.at[pl.ds(brow, ntail)],
                                cidx_v.at[0].at[pl.ds(0, ntail)])
                pltpu.sync_copy(row2.at[pl.ds(brow, ntail)],
                                ridx_v.at[0].at[pl.ds(0, ntail)])
                pltpu.sync_copy(val.at[pl.ds(base, ntail * SUB)],
                                val_v.at[0].at[pl.ds(0, ntail * SUB)])

                @pl.when(c == 1)
                def _():
                    def tbody(jj, _):
                        for l in range(SUB // LANE):
                            sl = pl.ds(l * LANE, LANE)
                            cidx_v[0, jj, sl] = cidx_v[0, jj, sl] + jnp.int32(n)
                        return 0

                    lax.fori_loop(0, ntail, tbody, 0)

                for j in range(ntail):
                    pltpu.async_copy(src.at[cidx_v.at[0].at[j]],
                                     rows_v.at[pl.ds(j * SUB, SUB)],
                                     gsem.at[j])
                for j in range(ntail):
                    pltpu.make_async_copy(src.at[cidx_v.at[0].at[0]],
                                          rows_v.at[pl.ds(j * SUB, SUB)],
                                          gsem.at[j]).wait()

                def tsbody(g, _):
                    vvec = val_v[0, pl.ds(g * LANE, LANE)]
                    for jj in range(LANE):
                        idx = g * LANE + jj
                        rows_v[idx, :] = rows_v[idx, :] * vvec[jj]
                    return 0

                lax.fori_loop(0, ntail * SUB // LANE, tsbody, 0)
                for j in range(ntail):
                    pltpu.sync_copy(rows_v.at[pl.ds(j * SUB, SUB)],
                                    acc_sh.at[ridx_v.at[0].at[j]], add=True)

        plsc.subcore_barrier()

        if not last:
            fire_idx(s, 0)

            def zfill2(i, _):
                rows_v[i, :] = zero
                return 0

            lax.fori_loop(0, zch, zfill2, 0)

            def ecopy_body(k, _):
                cid = s + k * NS

                @pl.when(cid < nzch)
                def _():
                    r0 = cid * zch
                    pltpu.sync_copy(acc_sh.at[pl.ds(r0, zch)],
                                    dst.at[pl.ds(cofs + r0, zch)])
                    pltpu.sync_copy(rows_v.at[pl.ds(0, zch)],
                                    acc_sh.at[pl.ds(r0, zch)])

                return 0

            lax.fori_loop(0, kzmax, ecopy_body, 0)
            plsc.subcore_barrier()

    run_layer(curA, curB, last=False, prefired=True)
    run_layer(curB, curC, last=False, prefired=True)
    run_layer(curC, None, last=True, prefired=True)

    def fin_body(k, _):
        cid = s + k * NS

        @pl.when(cid < nfch)
        def _():
            r0 = cid * FCH
            a = rows_v.at[pl.ds(0, FCH)]
            b = rows_v.at[pl.ds(256, FCH)]
            d = rows_v.at[pl.ds(512, FCH)]
            t = rows_v.at[pl.ds(768, FCH)]
            pltpu.sync_copy(curA.at[pl.ds(cofs + r0, FCH)], a)
            pltpu.sync_copy(curB.at[pl.ds(cofs + r0, FCH)], b)
            pltpu.sync_copy(curC.at[pl.ds(cofs + r0, FCH)], d)
            pltpu.sync_copy(acc_sh.at[pl.ds(r0, FCH)], t)

            def abody(i, _):
                t[i, :] = (a[i, :] + b[i, :]) + (d[i, :] + t[i, :])
                return 0

            lax.fori_loop(0, FCH, abody, 0)

            @pl.when(r0 < nu)
            def _():
                pltpu.sync_copy(
                    t, outU.at[pl.ds(r0, FCH), pl.ds(c * LANE, LANE)]
                )

            @pl.when(r0 >= nu)
            def _():
                pltpu.sync_copy(
                    t, outI.at[pl.ds(r0 - nu, FCH), pl.ds(c * LANE, LANE)]
                )

        return 0

    lax.fori_loop(0, kfmax, fin_body, 0)


@functools.partial(jax.jit, static_argnames=("n", "nu", "e", "ch", "zch"))
def _lightgcn_call(uE, iE, emask, col2, row2, val, *, n, nu, e, ch, zch):
    mesh = plsc.VectorSubcoreMesh(
        core_axis_name="c", subcore_axis_name="s", num_cores=NC, num_subcores=NS
    )
    d = NC * LANE
    body = functools.partial(_lightgcn_body, n=n, nu=nu, e=e, ch=ch, zch=zch)
    return pl.kernel(
        body,
        out_type=(
            jax.ShapeDtypeStruct((nu, d), jnp.float32),
            jax.ShapeDtypeStruct((n - nu, d), jnp.float32),
            jax.ShapeDtypeStruct((NC * n, LANE), jnp.float32),
            jax.ShapeDtypeStruct((NC * n, LANE), jnp.float32),
            jax.ShapeDtypeStruct((NC * n, LANE), jnp.float32),
        ),
        mesh=mesh,
        scratch_types=[
            pltpu.VMEM_SHARED((n, LANE), jnp.float32),
            pltpu.VMEM((2, ch // SUB, SUB), jnp.int32),
            pltpu.VMEM((2, ch // SUB, SUB), jnp.int32),
            pltpu.VMEM((2, ch), jnp.float32),
            pltpu.VMEM((RING * SUB, LANE), jnp.float32),
            pltpu.SemaphoreType.DMA((RING,)),
            pltpu.SemaphoreType.DMA((RING,)),
            pltpu.SemaphoreType.DMA((2,)),
        ],
        compiler_params=pltpu.CompilerParams(use_tc_tiling_on_sc=False),
    )(uE, iE, emask, col2, row2, val)


def kernel(edge_index, edge_values, uEmbeds, iEmbeds, adj_mask1, adj_mask2,
           emb_mask2):
    ch, zch = 2048, 1000
    nu = uEmbeds.shape[0]
    n = nu + iEmbeds.shape[0]
    e = edge_values.shape[0]
    assert uEmbeds.shape[1] == NC * LANE and ch % SUB == 0
    assert n % zch == 0 and zch % 8 == 0 and zch <= RING * SUB
    assert n % FCH == 0 and nu % FCH == 0 and FCH % 8 == 0 and 4 * FCH <= RING * SUB

    assert e % SUB == 0
    val = edge_values * adj_mask1 * adj_mask2
    col2 = edge_index[1].reshape(e // SUB, SUB)
    row2 = edge_index[0].reshape(e // SUB, SUB)

    outU, outI, _, _, _ = _lightgcn_call(
        uEmbeds, iEmbeds, emb_mask2, col2, row2, val, n=n, nu=nu, e=e, ch=ch,
        zch=zch
    )
    return outU, outI
